# baseline (speedup 1.0000x reference)
import jax
import jax.numpy as jnp
from jax import lax
from jax.experimental import pallas as pl
from jax.experimental.pallas import tpu as pltpu
from jax.experimental.pallas import tpu_sc as plsc

N = 10000
K = 16
D = 256

N_TC = 4880
N_SC = 5120

TC_BLOCK = 488
TAIL_BLOCK = 80
TAIL_OFF = N_TC // TAIL_BLOCK

NC = 2
NS = 16
NW = NC * NS
PER_W = N_SC // NW
CH = 8
NCHUNK = PER_W // CH


def _tc_main_body(src_ref, neigh_ref, wagg_ref, wself_ref, out_ref):
    neigh = neigh_ref[...]
    mean = jnp.sum(neigh, axis=1) * (1.0 / K)
    h = lax.dot_general(
        src_ref[...], wself_ref[...], (((1,), (0,)), ((), ())),
        preferred_element_type=jnp.float32,
    )
    h += lax.dot_general(
        mean, wagg_ref[...], (((1,), (0,)), ((), ())),
        preferred_element_type=jnp.float32,
    )
    out_ref[...] = jnp.maximum(h, 0.0)


def _tc_tail_body(src_ref, mean_ref, wagg_ref, wself_ref, out_ref):
    h = lax.dot_general(
        src_ref[...], wself_ref[...], (((1,), (0,)), ((), ())),
        preferred_element_type=jnp.float32,
    )
    h += lax.dot_general(
        mean_ref[...], wagg_ref[...], (((1,), (0,)), ((), ())),
        preferred_element_type=jnp.float32,
    )
    out_ref[...] = jnp.maximum(h, 0.0)


def _sc_mean_body(in_vmem, out_vmem):
    @pl.loop(0, CH)
    def _(nn):
        @pl.loop(0, D, step=32)
        def _(dc, nn=nn):
            for half in range(2):
                d0 = dc + 16 * half
                acc = in_vmem[nn, 0, pl.ds(d0, 16)]
                for k in range(1, K):
                    acc = acc + in_vmem[nn, k, pl.ds(d0, 16)]
                out_vmem[nn, pl.ds(d0, 16)] = acc * (1.0 / K)


def _sc_aggregate_part(neigh, start_row, rows):
    mesh = plsc.VectorSubcoreMesh(
        core_axis_name="c", subcore_axis_name="s", num_cores=1)
    start_blk = start_row // CH

    def body(neigh_hbm, out_hbm):
        pltpu.emit_pipeline(
            _sc_mean_body,
            grid=(rows // CH,),
            in_specs=[pl.BlockSpec(
                (CH, K, D), lambda i, o=start_blk: (i + o, 0, 0))],
            out_specs=[pl.BlockSpec((CH, D), lambda i: (i, 0))],
            core_axis_name=("c", "s"),
            dimension_semantics=(pltpu.PARALLEL,),
        )(neigh_hbm, out_hbm)

    f = pl.kernel(
        body,
        out_type=jax.ShapeDtypeStruct((rows, D), jnp.float32),
        mesh=mesh,
        scratch_types=[],
    )
    return f(neigh)


def _tc_tail_call(src, mean, W_agg, W_self, row_off, rows):
    off_blk = row_off // TAIL_BLOCK
    return pl.pallas_call(
        _tc_tail_body,
        grid=(rows // TAIL_BLOCK,),
        in_specs=[
            pl.BlockSpec((TAIL_BLOCK, D), lambda j, o=off_blk: (o + j, 0)),
            pl.BlockSpec((TAIL_BLOCK, D), lambda j: (j, 0)),
            pl.BlockSpec((D, D), lambda j: (0, 0)),
            pl.BlockSpec((D, D), lambda j: (0, 0)),
        ],
        out_specs=pl.BlockSpec((TAIL_BLOCK, D), lambda j: (j, 0)),
        out_shape=jax.ShapeDtypeStruct((rows, D), jnp.float32),
    )(src, mean, W_agg, W_self)


def kernel(src_node_features, neighbor_node_features, W_agg, W_self):
    half = N_SC // 2
    mean_a = _sc_aggregate_part(neighbor_node_features, N_TC, half)
    mean_b = _sc_aggregate_part(neighbor_node_features, N_TC + half, half)
    out_main = pl.pallas_call(
        _tc_main_body,
        grid=(N_TC // TC_BLOCK,),
        in_specs=[
            pl.BlockSpec((TC_BLOCK, D), lambda i: (i, 0)),
            pl.BlockSpec((TC_BLOCK, K, D), lambda i: (i, 0, 0)),
            pl.BlockSpec((D, D), lambda i: (0, 0)),
            pl.BlockSpec((D, D), lambda i: (0, 0)),
        ],
        out_specs=pl.BlockSpec((TC_BLOCK, D), lambda i: (i, 0)),
        out_shape=jax.ShapeDtypeStruct((N_TC, D), jnp.float32),
    )(src_node_features, neighbor_node_features, W_agg, W_self)
    out_a = _tc_tail_call(src_node_features, mean_a, W_agg, W_self, N_TC, half)
    out_b = _tc_tail_call(src_node_features, mean_b, W_agg, W_self,
                          N_TC + half, half)
    return jnp.concatenate([out_main, out_a, out_b], axis=0)


# speedup vs baseline: 1.8742x; 1.8742x over previous
import jax
import jax.numpy as jnp
from jax import lax
from jax.experimental import pallas as pl
from jax.experimental.pallas import tpu as pltpu
from jax.experimental.pallas import tpu_sc as plsc

N = 10000
K = 16
D = 256

N_TC = 8400
N_SC = 1600

TC_BLOCK = 840
TAIL_BLOCK = 80
CH = 8


def _tc_main_body(src_ref, neigh_ref, wagg_ref, wself_ref, out_ref):
    neigh = neigh_ref[...]
    mean = jnp.sum(neigh, axis=1) * (1.0 / K)
    h = lax.dot_general(
        src_ref[...], wself_ref[...], (((1,), (0,)), ((), ())),
        preferred_element_type=jnp.float32,
    )
    h += lax.dot_general(
        mean, wagg_ref[...], (((1,), (0,)), ((), ())),
        preferred_element_type=jnp.float32,
    )
    out_ref[...] = jnp.maximum(h, 0.0)


def _tc_tail_body(src_ref, mean_ref, wagg_ref, wself_ref, out_ref):
    h = lax.dot_general(
        src_ref[...], wself_ref[...], (((1,), (0,)), ((), ())),
        preferred_element_type=jnp.float32,
    )
    h += lax.dot_general(
        mean_ref[...], wagg_ref[...], (((1,), (0,)), ((), ())),
        preferred_element_type=jnp.float32,
    )
    out_ref[...] = jnp.maximum(h, 0.0)


def _sc_mean_body(in_vmem, out_vmem):
    @pl.loop(0, CH)
    def _(nn):
        @pl.loop(0, D, step=32)
        def _(dc, nn=nn):
            for half in range(2):
                d0 = dc + 16 * half
                acc = in_vmem[nn, 0, pl.ds(d0, 16)]
                for k in range(1, K):
                    acc = acc + in_vmem[nn, k, pl.ds(d0, 16)]
                out_vmem[nn, pl.ds(d0, 16)] = acc * (1.0 / K)


def _sc_aggregate_part(neigh, start_row, rows):
    mesh = plsc.VectorSubcoreMesh(core_axis_name="c", subcore_axis_name="s")
    start_blk = start_row // CH

    def body(neigh_hbm, out_hbm):
        pltpu.emit_pipeline(
            _sc_mean_body,
            grid=(rows // CH,),
            in_specs=[pl.BlockSpec(
                (CH, K, D), lambda i, o=start_blk: (i + o, 0, 0))],
            out_specs=[pl.BlockSpec((CH, D), lambda i: (i, 0))],
            core_axis_name=("c", "s"),
            dimension_semantics=(pltpu.PARALLEL,),
        )(neigh_hbm, out_hbm)

    f = pl.kernel(
        body,
        out_type=jax.ShapeDtypeStruct((rows, D), jnp.float32),
        mesh=mesh,
        scratch_types=[],
    )
    return f(neigh)


def _tc_tail_call(src, mean, W_agg, W_self, row_off, rows):
    off_blk = row_off // TAIL_BLOCK
    return pl.pallas_call(
        _tc_tail_body,
        grid=(rows // TAIL_BLOCK,),
        in_specs=[
            pl.BlockSpec((TAIL_BLOCK, D), lambda j, o=off_blk: (o + j, 0)),
            pl.BlockSpec((TAIL_BLOCK, D), lambda j: (j, 0)),
            pl.BlockSpec((D, D), lambda j: (0, 0)),
            pl.BlockSpec((D, D), lambda j: (0, 0)),
        ],
        out_specs=pl.BlockSpec((TAIL_BLOCK, D), lambda j: (j, 0)),
        out_shape=jax.ShapeDtypeStruct((rows, D), jnp.float32),
    )(src, mean, W_agg, W_self)


def kernel(src_node_features, neighbor_node_features, W_agg, W_self):
    mean_sc = _sc_aggregate_part(neighbor_node_features, N_TC, N_SC)
    out_main = pl.pallas_call(
        _tc_main_body,
        grid=(N_TC // TC_BLOCK,),
        in_specs=[
            pl.BlockSpec((TC_BLOCK, D), lambda i: (i, 0)),
            pl.BlockSpec((TC_BLOCK, K, D), lambda i: (i, 0, 0)),
            pl.BlockSpec((D, D), lambda i: (0, 0)),
            pl.BlockSpec((D, D), lambda i: (0, 0)),
        ],
        out_specs=pl.BlockSpec((TC_BLOCK, D), lambda i: (i, 0)),
        out_shape=jax.ShapeDtypeStruct((N_TC, D), jnp.float32),
    )(src_node_features, neighbor_node_features, W_agg, W_self)
    out_tail = _tc_tail_call(src_node_features, mean_sc, W_agg, W_self,
                             N_TC, N_SC)
    return jnp.concatenate([out_main, out_tail], axis=0)


# speedup vs baseline: 2.8739x; 1.5334x over previous
import jax
import jax.numpy as jnp
from jax import lax
from jax.experimental import pallas as pl

N = 10000
K = 16
D_IN = 256
D_OUT = 256
BLOCK = 1000


def _fused_kernel(src_ref, na_ref, nb_ref, wagg_ref, wself_ref, out_ref):
    s = jnp.sum(na_ref[...], axis=1) + jnp.sum(nb_ref[...], axis=1)
    mean = s * (1.0 / K)
    h = lax.dot_general(
        src_ref[...], wself_ref[...], (((1,), (0,)), ((), ())),
        preferred_element_type=jnp.float32,
    )
    h += lax.dot_general(
        mean, wagg_ref[...], (((1,), (0,)), ((), ())),
        preferred_element_type=jnp.float32,
    )
    out_ref[...] = jnp.maximum(h, 0.0)


def kernel(src_node_features, neighbor_node_features, W_agg, W_self):
    n = src_node_features.shape[0]
    grid = (n // BLOCK,)
    return pl.pallas_call(
        _fused_kernel,
        grid=grid,
        in_specs=[
            pl.BlockSpec((BLOCK, D_IN), lambda i: (i, 0)),
            pl.BlockSpec((BLOCK, K // 2, D_IN), lambda i: (i, 0, 0)),
            pl.BlockSpec((BLOCK, K // 2, D_IN), lambda i: (i, 1, 0)),
            pl.BlockSpec((D_IN, D_OUT), lambda i: (0, 0)),
            pl.BlockSpec((D_IN, D_OUT), lambda i: (0, 0)),
        ],
        out_specs=pl.BlockSpec((BLOCK, D_OUT), lambda i: (i, 0)),
        out_shape=jax.ShapeDtypeStruct((n, D_OUT), jnp.float32),
    )(src_node_features, neighbor_node_features, neighbor_node_features,
      W_agg, W_self)


# speedup vs baseline: 3.0897x; 1.0751x over previous
import jax
import jax.numpy as jnp
from jax import lax
from jax.experimental import pallas as pl

N = 10000
K = 16
D_IN = 256
D_OUT = 256
BLOCK = 1000


def _fused_kernel(src_ref, neigh_ref, wagg_ref, wself_ref, out_ref):
    neigh = neigh_ref[...]
    mean = jnp.sum(neigh, axis=1) * (1.0 / K)
    h = lax.dot_general(
        src_ref[...], wself_ref[...], (((1,), (0,)), ((), ())),
        preferred_element_type=jnp.float32,
    )
    h += lax.dot_general(
        mean, wagg_ref[...], (((1,), (0,)), ((), ())),
        preferred_element_type=jnp.float32,
    )
    out_ref[...] = jnp.maximum(h, 0.0)


def kernel(src_node_features, neighbor_node_features, W_agg, W_self):
    n = src_node_features.shape[0]
    grid = (n // BLOCK,)
    return pl.pallas_call(
        _fused_kernel,
        grid=grid,
        in_specs=[
            pl.BlockSpec((BLOCK, D_IN), lambda i: (i, 0)),
            pl.BlockSpec((BLOCK, K, D_IN), lambda i: (i, 0, 0)),
            pl.BlockSpec((D_IN, D_OUT), lambda i: (0, 0)),
            pl.BlockSpec((D_IN, D_OUT), lambda i: (0, 0)),
        ],
        out_specs=pl.BlockSpec((BLOCK, D_OUT), lambda i: (i, 0)),
        out_shape=jax.ShapeDtypeStruct((n, D_OUT), jnp.float32),
    )(src_node_features, neighbor_node_features, W_agg, W_self)
